# Initial kernel scaffold; baseline (speedup 1.0000x reference)
import functools

import numpy as np
import jax
import jax.numpy as jnp
from jax import lax
from jax.experimental import pallas as pl
from jax.experimental.pallas import tpu as pltpu
from jax.experimental.pallas import tpu_sc as plsc

EMBED_DIM = 128
N_NODES = 10000
N_EDGES = 320000
NUM_RELS = 100
BATCH = 1024
NUM_H = 4

PAD = 80
EP2 = 2 * PAD
TROWS = 10112
ACC_ROWS = 10240
RO_ROWS = 1280
X2_COLS = NUM_H * EMBED_DIM

NC, NS = 2, 16
NW = NC * NS
EPW = N_EDGES // NW
EB = 80
NBLK = EPW // EB


def _dft_mats():
    n = EMBED_DIM
    j = np.arange(n)
    ki = np.arange(n // 2 + 1)
    ang = 2.0 * np.pi / n * np.outer(j, ki)
    w = np.where((ki == 0) | (2 * ki == n), 1.0, 2.0)
    iang = 2.0 * np.pi / n * np.outer(ki, j)
    f2 = np.zeros((n, EP2), np.float32)
    f2[:, : n // 2 + 1] = np.cos(ang)
    f2[:, PAD : PAD + n // 2 + 1] = -np.sin(ang)
    g2 = np.zeros((EP2, n), np.float32)
    g2[: n // 2 + 1] = (w[:, None] * np.cos(iang)) / n
    g2[PAD : PAD + n // 2 + 1] = -(w[:, None] * np.sin(iang)) / n
    return f2, g2


_F2P, _G2P = _dft_mats()


def _spectrum_body(x_ref, f_ref, o_ref):
    o_ref[...] = jnp.dot(x_ref[...], f_ref[...],
                         preferred_element_type=jnp.float32)


def _kmat_body(g_ref, w_ref, o_ref):
    o_ref[0] = jnp.dot(g_ref[...], w_ref[0],
                       preferred_element_type=jnp.float32)


def _mid_body(a_ref, k_ref, b_ref, f_ref, o_ref):
    agg = a_ref[0] + a_ref[1]
    z = jnp.maximum(
        jnp.dot(agg, k_ref[0], preferred_element_type=jnp.float32)
        + b_ref[0], 0.0)
    o_ref[0] = jnp.dot(z, f_ref[...], preferred_element_type=jnp.float32)


def _out_body(a_ref, k_ref, b_ref, o_ref):
    agg = a_ref[0, 0] + a_ref[0, 1]
    o_ref[...] = jnp.maximum(
        jnp.dot(agg, k_ref[0], preferred_element_type=jnp.float32)
        + b_ref[0], 0.0)


def _final_body(a_ref, o_ref):
    o_ref[...] = a_ref[0] + a_ref[1]


def _tc_spectrum(x):
    m = x.shape[0]
    return pl.pallas_call(
        _spectrum_body,
        grid=(m // 128,),
        in_specs=[pl.BlockSpec((128, EMBED_DIM), lambda i: (i, 0)),
                  pl.BlockSpec((EMBED_DIM, EP2), lambda i: (0, 0))],
        out_specs=pl.BlockSpec((128, EP2), lambda i: (i, 0)),
        out_shape=jax.ShapeDtypeStruct((m, EP2), jnp.float32),
    )(x, jnp.asarray(_F2P))


def _tc_kmats(wf):
    return pl.pallas_call(
        _kmat_body,
        grid=(8,),
        in_specs=[pl.BlockSpec((EP2, EMBED_DIM), lambda i: (0, 0)),
                  pl.BlockSpec((1, EMBED_DIM, EMBED_DIM), lambda i: (i, 0, 0))],
        out_specs=pl.BlockSpec((1, EP2, EMBED_DIM), lambda i: (i, 0, 0)),
        out_shape=jax.ShapeDtypeStruct((8, EP2, EMBED_DIM), jnp.float32),
    )(jnp.asarray(_G2P), wf)


def _tc_layer_mid(acc1, kmats, bf):
    return pl.pallas_call(
        _mid_body,
        grid=(NUM_H, TROWS // 128),
        in_specs=[
            pl.BlockSpec((2, 128, EP2), lambda h, m: (0, m, 0)),
            pl.BlockSpec((1, EP2, EMBED_DIM), lambda h, m: (2 * h, 0, 0)),
            pl.BlockSpec((1, 1, EMBED_DIM), lambda h, m: (2 * h, 0, 0)),
            pl.BlockSpec((EMBED_DIM, EP2), lambda h, m: (0, 0)),
        ],
        out_specs=pl.BlockSpec((1, 128, EP2), lambda h, m: (h, m, 0)),
        out_shape=jax.ShapeDtypeStruct((NUM_H, TROWS, EP2), jnp.float32),
    )(acc1, kmats, bf, jnp.asarray(_F2P))


def _tc_layer_out(acc2s, kmats, bf):
    return pl.pallas_call(
        _out_body,
        grid=(NUM_H, ACC_ROWS // 128),
        in_specs=[
            pl.BlockSpec((1, 2, 128, EP2), lambda h, m: (h, 0, m, 0)),
            pl.BlockSpec((1, EP2, EMBED_DIM), lambda h, m: (2 * h + 1, 0, 0)),
            pl.BlockSpec((1, 1, EMBED_DIM), lambda h, m: (2 * h + 1, 0, 0)),
        ],
        out_specs=pl.BlockSpec((128, EMBED_DIM), lambda h, m: (m, h)),
        out_shape=jax.ShapeDtypeStruct((ACC_ROWS, X2_COLS), jnp.float32),
    )(acc2s, kmats, bf)


def _tc_final_add(accr):
    return pl.pallas_call(
        _final_body,
        grid=(BATCH // 128,),
        in_specs=[pl.BlockSpec((2, 128, X2_COLS), lambda m: (0, m, 0))],
        out_specs=pl.BlockSpec((128, X2_COLS), lambda m: (m, 0)),
        out_shape=jax.ShapeDtypeStruct((BATCH, X2_COLS), jnp.float32),
    )(accr)


def _edge_body(tab, rtab, srcr, typr, dstr, out,
               sidx, tidx, didx, xrows, rrows, prod, zbuf, acc, sem):
    c = lax.axis_index("c")
    s = lax.axis_index("s")
    wid = s * NC + c
    z16 = jnp.zeros((16,), jnp.float32)

    def zb(i, carry):
        for q in range(EP2 // 16):
            zbuf[i, pl.ds(q * 16, 16)] = z16
        return carry
    lax.fori_loop(0, 64, zb, 0)

    rows_per_sub = ACC_ROWS // NS

    def za(i, carry):
        pltpu.sync_copy(zbuf, acc.at[pl.ds(s * rows_per_sub + i * 64, 64)])
        return carry
    lax.fori_loop(0, rows_per_sub // 64, za, 0)
    plsc.subcore_barrier()

    ebase = wid * EPW

    def blk(k, carry):
        off = ebase + k * EB
        pltpu.sync_copy(srcr.at[pl.ds(off, EB)], sidx)
        pltpu.sync_copy(typr.at[pl.ds(off, EB)], tidx)
        pltpu.sync_copy(dstr.at[pl.ds(off, EB)], didx)
        cp1 = pltpu.async_copy(tab.at[sidx], xrows, sem)
        cp2 = pltpu.async_copy(rtab.at[tidx], rrows, sem)
        cp1.wait()
        cp2.wait()

        def pe(e, cc):
            for q in range(PAD // 16):
                xr = xrows[e, pl.ds(q * 16, 16)]
                xi = xrows[e, pl.ds(PAD + q * 16, 16)]
                rr = rrows[e, pl.ds(q * 16, 16)]
                ri = rrows[e, pl.ds(PAD + q * 16, 16)]
                prod[e, pl.ds(q * 16, 16)] = xr * rr + xi * ri
                prod[e, pl.ds(PAD + q * 16, 16)] = xr * ri - xi * rr
            return cc
        lax.fori_loop(0, EB, pe, 0)
        pltpu.sync_copy(prod, acc.at[didx], add=True)
        return carry
    lax.fori_loop(0, NBLK, blk, 0)
    plsc.subcore_barrier()

    def co(i, carry):
        r0 = s * rows_per_sub + i * 64
        pltpu.sync_copy(acc.at[pl.ds(r0, 64)], out.at[c, pl.ds(r0, 64)])
        return carry
    lax.fori_loop(0, rows_per_sub // 64, co, 0)


def _readout_body(x2r, bidxr, out, bidx_v, rows_v, zbuf, acc, sem):
    c = lax.axis_index("c")
    s = lax.axis_index("s")
    wid = s * NC + c
    z16 = jnp.zeros((16,), jnp.float32)

    def zb(i, carry):
        for q in range(X2_COLS // 16):
            zbuf[i, pl.ds(q * 16, 16)] = z16
        return carry
    lax.fori_loop(0, 16, zb, 0)

    rows_per_sub = RO_ROWS // NS

    def za(i, carry):
        pltpu.sync_copy(zbuf, acc.at[pl.ds(s * rows_per_sub + i * 16, 16)])
        return carry
    lax.fori_loop(0, rows_per_sub // 16, za, 0)
    plsc.subcore_barrier()

    rows_per_w = ACC_ROWS // NW

    def blk(k, carry):
        off = wid * rows_per_w + k * EB
        pltpu.sync_copy(x2r.at[pl.ds(off, EB)], rows_v)
        pltpu.sync_copy(bidxr.at[pl.ds(off, EB)], bidx_v)
        pltpu.sync_copy(rows_v, acc.at[bidx_v], add=True)
        return carry
    lax.fori_loop(0, rows_per_w // EB, blk, 0)
    plsc.subcore_barrier()

    cop_rows = BATCH // NS
    pltpu.sync_copy(acc.at[pl.ds(s * cop_rows, cop_rows)],
                    out.at[c, pl.ds(s * cop_rows, cop_rows)])


@functools.lru_cache(maxsize=None)
def _build_sc_kernels():
    mesh = plsc.VectorSubcoreMesh(core_axis_name="c", subcore_axis_name="s")
    edge = pl.kernel(
        _edge_body, mesh=mesh,
        out_type=jax.ShapeDtypeStruct((NC, ACC_ROWS, EP2), jnp.float32),
        scratch_types=[
            pltpu.VMEM((EB,), jnp.int32),
            pltpu.VMEM((EB,), jnp.int32),
            pltpu.VMEM((EB,), jnp.int32),
            pltpu.VMEM((EB, EP2), jnp.float32),
            pltpu.VMEM((EB, EP2), jnp.float32),
            pltpu.VMEM((EB, EP2), jnp.float32),
            pltpu.VMEM((64, EP2), jnp.float32),
            pltpu.VMEM_SHARED((ACC_ROWS, EP2), jnp.float32),
            pltpu.SemaphoreType.DMA,
        ],
    )
    readout = pl.kernel(
        _readout_body, mesh=mesh,
        out_type=jax.ShapeDtypeStruct((NC, BATCH, X2_COLS), jnp.float32),
        scratch_types=[
            pltpu.VMEM((EB,), jnp.int32),
            pltpu.VMEM((EB, X2_COLS), jnp.float32),
            pltpu.VMEM((16, X2_COLS), jnp.float32),
            pltpu.VMEM_SHARED((RO_ROWS, X2_COLS), jnp.float32),
            pltpu.SemaphoreType.DMA,
        ],
    )
    return edge, readout


def kernel(ent_e, edge_index, edge_type, batch_idx, rel_table, W, b):
    edge_pass, readout = _build_sc_kernels()

    src = edge_index[0]
    dst = edge_index[1]
    typ10 = edge_type + N_NODES

    tin = jnp.concatenate(
        [ent_e, rel_table,
         jnp.zeros((TROWS - N_NODES - NUM_RELS, EMBED_DIM), jnp.float32)],
        axis=0)
    t1 = _tc_spectrum(tin)
    kmats = _tc_kmats(W.reshape(8, EMBED_DIM, EMBED_DIM))
    bf = b.reshape(8, 1, EMBED_DIM)

    acc1 = edge_pass(t1, t1, src, typ10, dst)
    mid = _tc_layer_mid(acc1[:, :TROWS], kmats, bf)

    acc2 = [edge_pass(mid[h], t1, src, typ10, dst) for h in range(NUM_H)]
    acc2s = jnp.stack(acc2, axis=0)

    x2 = _tc_layer_out(acc2s, kmats, bf)
    bidx_pad = jnp.concatenate(
        [batch_idx,
         BATCH + (jnp.arange(ACC_ROWS - N_NODES, dtype=jnp.int32) % 256)])
    accr = readout(x2, bidx_pad)
    return _tc_final_add(accr)


# speedup vs baseline: 2.7047x; 2.7047x over previous
import functools

import numpy as np
import jax
import jax.numpy as jnp
from jax import lax
from jax.experimental import pallas as pl
from jax.experimental.pallas import tpu as pltpu
from jax.experimental.pallas import tpu_sc as plsc

EMBED_DIM = 128
N_NODES = 10000
N_EDGES = 320000
NUM_RELS = 100
BATCH = 1024
NUM_H = 4

HW = 64
EP2 = 2 * HW
TROWS = 10112
ACC_ROWS = 10240
RO_ROWS = 1280
X2_COLS = NUM_H * EMBED_DIM

NC, NS = 2, 16
NW = NC * NS
EPW = N_EDGES // NW
EB = 80
NBLK = EPW // EB


def _dft_mats():
    n = EMBED_DIM
    j = np.arange(n)
    ki = np.arange(n // 2 + 1)
    ang = 2.0 * np.pi / n * np.outer(j, ki)
    fc, fs = np.cos(ang), np.sin(ang)
    w = np.where((ki == 0) | (2 * ki == n), 1.0, 2.0)
    iang = 2.0 * np.pi / n * np.outer(ki, j)
    gr = (w[:, None] * np.cos(iang)) / n
    gi = (w[:, None] * np.sin(iang)) / n
    f2 = np.zeros((n, n), np.float32)
    f2[:, :HW] = fc[:, :HW]
    f2[:, HW] = fc[:, HW]
    f2[:, HW + 1:] = -fs[:, 1:HW]
    g2 = np.zeros((n, n), np.float32)
    g2[:HW] = gr[:HW]
    g2[HW] = gr[HW]
    g2[HW + 1:] = -gi[1:HW]
    return f2, g2


_F2P, _G2P = _dft_mats()


def _spectrum_body(x_ref, f_ref, o_ref):
    o_ref[...] = jnp.dot(x_ref[...], f_ref[...],
                         preferred_element_type=jnp.float32)


def _kmat_body(g_ref, w_ref, o_ref):
    o_ref[0] = jnp.dot(g_ref[...], w_ref[0],
                       preferred_element_type=jnp.float32)


def _mid_body(a_ref, k_ref, b_ref, f_ref, o_ref):
    agg = a_ref[0] + a_ref[1]
    z = jnp.maximum(
        jnp.dot(agg, k_ref[0], preferred_element_type=jnp.float32)
        + b_ref[0], 0.0)
    o_ref[0] = jnp.dot(z, f_ref[...], preferred_element_type=jnp.float32)


def _out_body(a_ref, k_ref, b_ref, o_ref):
    agg = a_ref[0, 0] + a_ref[0, 1]
    o_ref[0] = jnp.maximum(
        jnp.dot(agg, k_ref[0], preferred_element_type=jnp.float32)
        + b_ref[0], 0.0)


def _final_body(a_ref, o_ref):
    o_ref[...] = a_ref[0, 0] + a_ref[1, 0]


def _tc_spectrum(x):
    m = x.shape[0]
    return pl.pallas_call(
        _spectrum_body,
        grid=(m // 128,),
        in_specs=[pl.BlockSpec((128, EMBED_DIM), lambda i: (i, 0)),
                  pl.BlockSpec((EMBED_DIM, EP2), lambda i: (0, 0))],
        out_specs=pl.BlockSpec((128, EP2), lambda i: (i, 0)),
        out_shape=jax.ShapeDtypeStruct((m, EP2), jnp.float32),
    )(x, jnp.asarray(_F2P))


def _tc_kmats(wf):
    return pl.pallas_call(
        _kmat_body,
        grid=(8,),
        in_specs=[pl.BlockSpec((EP2, EMBED_DIM), lambda i: (0, 0)),
                  pl.BlockSpec((1, EMBED_DIM, EMBED_DIM), lambda i: (i, 0, 0))],
        out_specs=pl.BlockSpec((1, EP2, EMBED_DIM), lambda i: (i, 0, 0)),
        out_shape=jax.ShapeDtypeStruct((8, EP2, EMBED_DIM), jnp.float32),
    )(jnp.asarray(_G2P), wf)


def _tc_layer_mid(acc1, kmats, bf):
    return pl.pallas_call(
        _mid_body,
        grid=(NUM_H, TROWS // 128),
        in_specs=[
            pl.BlockSpec((2, 128, EP2), lambda h, m: (0, m, 0)),
            pl.BlockSpec((1, EP2, EMBED_DIM), lambda h, m: (2 * h, 0, 0)),
            pl.BlockSpec((1, 1, EMBED_DIM), lambda h, m: (2 * h, 0, 0)),
            pl.BlockSpec((EMBED_DIM, EP2), lambda h, m: (0, 0)),
        ],
        out_specs=pl.BlockSpec((1, 128, EP2), lambda h, m: (h, m, 0)),
        out_shape=jax.ShapeDtypeStruct((NUM_H, TROWS, EP2), jnp.float32),
    )(acc1, kmats, bf, jnp.asarray(_F2P))


def _tc_layer_out(acc2s, kmats, bf):
    return pl.pallas_call(
        _out_body,
        grid=(NUM_H, ACC_ROWS // 128),
        in_specs=[
            pl.BlockSpec((1, 2, 128, EP2), lambda h, m: (h, 0, m, 0)),
            pl.BlockSpec((1, EP2, EMBED_DIM), lambda h, m: (2 * h + 1, 0, 0)),
            pl.BlockSpec((1, 1, EMBED_DIM), lambda h, m: (2 * h + 1, 0, 0)),
        ],
        out_specs=pl.BlockSpec((1, 128, EMBED_DIM), lambda h, m: (h, m, 0)),
        out_shape=jax.ShapeDtypeStruct((NUM_H, ACC_ROWS, EMBED_DIM),
                                       jnp.float32),
    )(acc2s, kmats, bf)


def _tc_final_add(accr):
    return pl.pallas_call(
        _final_body,
        grid=(NUM_H, BATCH // 128),
        in_specs=[pl.BlockSpec((2, 1, 128, EMBED_DIM),
                               lambda h, m: (0, h, m, 0))],
        out_specs=pl.BlockSpec((128, EMBED_DIM), lambda h, m: (m, h)),
        out_shape=jax.ShapeDtypeStruct((BATCH, X2_COLS), jnp.float32),
    )(accr)


def _edge_body(tab, rtab, srcr, typr, dstr, out,
               sidx, tidx, didx, xrows, rrows, prod, zbuf, acc, sem):
    c = lax.axis_index("c")
    s = lax.axis_index("s")
    wid = s * NC + c
    z16 = jnp.zeros((16,), jnp.float32)
    m0 = lax.broadcasted_iota(jnp.int32, (16,), 0) == 0

    def zb(i, carry):
        for q in range(EP2 // 16):
            zbuf[i, pl.ds(q * 16, 16)] = z16
        return carry
    lax.fori_loop(0, 64, zb, 0)

    rows_per_sub = ACC_ROWS // NS

    def za(i, carry):
        pltpu.sync_copy(zbuf, acc.at[pl.ds(s * rows_per_sub + i * 64, 64)])
        return carry
    lax.fori_loop(0, rows_per_sub // 64, za, 0)
    plsc.subcore_barrier()

    ebase = wid * EPW

    def blk(k, carry):
        off = ebase + k * EB
        pltpu.sync_copy(srcr.at[pl.ds(off, EB)], sidx)
        pltpu.sync_copy(typr.at[pl.ds(off, EB)], tidx)
        pltpu.sync_copy(dstr.at[pl.ds(off, EB)], didx)
        cp1 = pltpu.async_copy(tab.at[sidx], xrows, sem)
        cp2 = pltpu.async_copy(rtab.at[tidx], rrows, sem)
        cp1.wait()
        cp2.wait()

        def pe(e, cc):
            for q in range(HW // 16):
                xa = xrows[e, pl.ds(q * 16, 16)]
                xb = xrows[e, pl.ds(HW + q * 16, 16)]
                ra = rrows[e, pl.ds(q * 16, 16)]
                rb = rrows[e, pl.ds(HW + q * 16, 16)]
                t1 = xa * ra
                t2 = xb * rb
                pre = t1 + t2
                pim = xa * rb - xb * ra
                if q == 0:
                    pre = jnp.where(m0, t1, pre)
                    pim = jnp.where(m0, t2, pim)
                prod[e, pl.ds(q * 16, 16)] = pre
                prod[e, pl.ds(HW + q * 16, 16)] = pim
            return cc
        lax.fori_loop(0, EB, pe, 0)
        pltpu.async_copy(prod, acc.at[didx], sem, add=True).wait()
        return carry
    lax.fori_loop(0, NBLK, blk, 0)
    plsc.subcore_barrier()

    def co(i, carry):
        r0 = s * rows_per_sub + i * 64
        pltpu.sync_copy(acc.at[pl.ds(r0, 64)], out.at[c, pl.ds(r0, 64)])
        return carry
    lax.fori_loop(0, rows_per_sub // 64, co, 0)


def _readout_body(x2r, bidxr, out, bidx_v, hidx_v, rows_v, zbuf, acc, sem):
    c = lax.axis_index("c")
    s = lax.axis_index("s")
    wid = s * NC + c
    z16 = jnp.zeros((16,), jnp.float32)

    def zb(i, carry):
        for q in range(EMBED_DIM // 16):
            zbuf[i, pl.ds(q * 16, 16)] = z16
        return carry
    lax.fori_loop(0, 64, zb, 0)

    rows_per_sub = NUM_H * RO_ROWS // NS

    def za(i, carry):
        pltpu.sync_copy(zbuf, acc.at[pl.ds(s * rows_per_sub + i * 64, 64)])
        return carry
    lax.fori_loop(0, rows_per_sub // 64, za, 0)
    plsc.subcore_barrier()

    rows_per_w = ACC_ROWS // NW

    def blk(k, carry):
        off = wid * rows_per_w + k * EB
        pltpu.sync_copy(bidxr.at[pl.ds(off, EB)], bidx_v)
        for h in range(NUM_H):
            pltpu.sync_copy(x2r.at[h, pl.ds(off, EB)], rows_v)
            for j in range(EB // 16):
                hidx_v[pl.ds(j * 16, 16)] = (
                    bidx_v[pl.ds(j * 16, 16)] + h * RO_ROWS)
            pltpu.async_copy(rows_v, acc.at[hidx_v], sem, add=True).wait()
        return carry
    lax.fori_loop(0, rows_per_w // EB, blk, 0)
    plsc.subcore_barrier()

    cop_rows = BATCH // NS
    for h in range(NUM_H):
        pltpu.sync_copy(
            acc.at[pl.ds(h * RO_ROWS + s * cop_rows, cop_rows)],
            out.at[c, pl.ds(h * BATCH + s * cop_rows, cop_rows)])


@functools.lru_cache(maxsize=None)
def _build_sc_kernels():
    mesh = plsc.VectorSubcoreMesh(core_axis_name="c", subcore_axis_name="s")
    edge = pl.kernel(
        _edge_body, mesh=mesh,
        out_type=jax.ShapeDtypeStruct((NC, ACC_ROWS, EP2), jnp.float32),
        scratch_types=[
            pltpu.VMEM((EB,), jnp.int32),
            pltpu.VMEM((EB,), jnp.int32),
            pltpu.VMEM((EB,), jnp.int32),
            pltpu.VMEM((EB, EP2), jnp.float32),
            pltpu.VMEM((EB, EP2), jnp.float32),
            pltpu.VMEM((EB, EP2), jnp.float32),
            pltpu.VMEM((64, EP2), jnp.float32),
            pltpu.VMEM_SHARED((ACC_ROWS, EP2), jnp.float32),
            pltpu.SemaphoreType.DMA,
        ],
    )
    readout = pl.kernel(
        _readout_body, mesh=mesh,
        out_type=jax.ShapeDtypeStruct((NC, NUM_H * BATCH, EMBED_DIM),
                                      jnp.float32),
        scratch_types=[
            pltpu.VMEM((EB,), jnp.int32),
            pltpu.VMEM((EB,), jnp.int32),
            pltpu.VMEM((EB, EMBED_DIM), jnp.float32),
            pltpu.VMEM((64, EMBED_DIM), jnp.float32),
            pltpu.VMEM_SHARED((NUM_H * RO_ROWS, EMBED_DIM), jnp.float32),
            pltpu.SemaphoreType.DMA,
        ],
    )
    return edge, readout


def kernel(ent_e, edge_index, edge_type, batch_idx, rel_table, W, b):
    edge_pass, readout = _build_sc_kernels()

    src = edge_index[0]
    dst = edge_index[1]
    typ10 = edge_type + N_NODES

    tin = jnp.concatenate(
        [ent_e, rel_table,
         jnp.zeros((TROWS - N_NODES - NUM_RELS, EMBED_DIM), jnp.float32)],
        axis=0)
    t1 = _tc_spectrum(tin)
    kmats = _tc_kmats(W.reshape(8, EMBED_DIM, EMBED_DIM))
    bf = b.reshape(8, 1, EMBED_DIM)

    acc1 = edge_pass(t1, t1, src, typ10, dst)
    mid = _tc_layer_mid(acc1[:, :TROWS], kmats, bf)

    acc2 = [edge_pass(mid[h], t1, src, typ10, dst) for h in range(NUM_H)]
    acc2s = jnp.stack(acc2, axis=0)

    x2 = _tc_layer_out(acc2s, kmats, bf)
    bidx_pad = jnp.concatenate(
        [batch_idx,
         BATCH + (jnp.arange(ACC_ROWS - N_NODES, dtype=jnp.int32) % 256)])
    accr = readout(x2, bidx_pad)
    return _tc_final_add(accr.reshape(NC, NUM_H, BATCH, EMBED_DIM))
